# baseline (speedup 1.0000x reference)
import jax
import jax.numpy as jnp
from jax.experimental import pallas as pl
from jax.experimental.pallas import tpu as pltpu

_B, _C, _H, _W = 4, 768, 224, 224
_HP, _WP = _H // 2, _W // 2
_WPAD = 128
_HWP = _HP * _WPAD
_HW = _HP * _WP
_E, _RED = 16, 48
_G1, _G2 = 8, 4
_EPS = 1e-5
_CB = 16
_S = _HWP // 7
_NS = _HWP // _S


def _pass1_body(x_ref, dw_ref, ph_ref, pw_ref, h_ref, sum_ref, ssq_ref):
    xb = x_ref[0]
    phb = jnp.broadcast_to(ph_ref[...], (_CB, _HP, _H))
    xh = jax.lax.dot_general(
        phb, xb, (((2,), (1,)), ((0,), (0,))),
        preferred_element_type=jnp.float32,
    ).astype(jnp.bfloat16)
    r = jax.lax.dot_general(
        xh.reshape(_CB * _HP, _W), pw_ref[...],
        (((1,), (0,)), ((), ())),
        preferred_element_type=jnp.float32,
    ).reshape(_CB, _HP, 3 * _WPAD)

    w = dw_ref[...]
    rows = []
    for dy in (-1, 0, 1):
        acc = jnp.zeros((_CB, _HP, _WPAD), jnp.float32)
        for t in range(3):
            k = 3 * (dy + 1) + t
            q = r[:, :, t * _WPAD:(t + 1) * _WPAD]
            acc = acc + q * w[:, k:k + 1].reshape(_CB, 1, 1)
        rows.append(acc)

    ri = jax.lax.broadcasted_iota(jnp.int32, (1, _HP, 1), 1)
    h = rows[1]
    h = h + jnp.roll(rows[0], 1, axis=1) * (ri >= 1).astype(jnp.float32)
    h = h + jnp.roll(rows[2], -1, axis=1) * (ri < _HP - 1).astype(jnp.float32)

    h_ref[0] = h.astype(jnp.bfloat16)
    cb = pl.program_id(1)
    sum_ref[0, pl.ds(cb * _CB, _CB), :] = jnp.sum(h, axis=(1, 2)).reshape(_CB, 1)
    ssq_ref[0, pl.ds(cb * _CB, _CB), :] = jnp.sum(h * h, axis=(1, 2)).reshape(_CB, 1)


def _pass2_body(h_ref, sum_ref, ssq_ref, g1w_ref, g1b_ref, w1_ref,
                gm1_ref, gm1t_ref, g2w_ref, g2b_ref, w2_ref, b2_ref,
                gm2_ref, gm2t_ref, h2_ref, coef_ref, vals_ref, idx_ref):
    s = pl.program_id(1)

    def _mm(a, b):
        return jax.lax.dot_general(a, b, (((1,), (0,)), ((), ())),
                                   preferred_element_type=jnp.float32)

    @pl.when(s == 0)
    def _():
        n1 = float((_C // _G1) * _HW)
        mu_g = _mm(gm1t_ref[...], sum_ref[0]) * (1.0 / n1)
        var_g = _mm(gm1t_ref[...], ssq_ref[0]) * (1.0 / n1) - mu_g * mu_g
        rs_g = jax.lax.rsqrt(var_g + _EPS)
        mu_c = _mm(gm1_ref[...], mu_g)
        rs_c = _mm(gm1_ref[...], rs_g)
        a_c = g1w_ref[...] * rs_c
        b_c = g1b_ref[...] - mu_c * a_c
        coef_ref[0] = jnp.concatenate([a_c, b_c], axis=1)

    ab = coef_ref[0]
    a_c = ab[:, 0:1, None]
    b_c = ab[:, 1:2, None]
    h = h_ref[0].astype(jnp.float32)
    a1 = h * a_c + b_c
    a1 = a1 * (0.5 * jnp.tanh(a1 * 0.5) + 0.5)
    li = jax.lax.broadcasted_iota(jnp.int32, (1, 1, _WPAD), 2)
    cmask = (li < _WP).astype(jnp.float32)
    chunk = jax.lax.dot_general(w1_ref[...], a1, (((1,), (0,)), ((), ())),
                                preferred_element_type=jnp.float32)
    _RB = _HP // _NS
    h2_ref[0, :, pl.ds(s * _RB, _RB), :] = chunk * cmask

    @pl.when(s == _NS - 1)
    def _():
        h2 = h2_ref[0]
        n2 = float((_RED // _G2) * _HW)
        csum = jnp.sum(h2, axis=(1, 2)).reshape(_RED, 1)
        cssq = jnp.sum(h2 * h2, axis=(1, 2)).reshape(_RED, 1)
        mu2 = _mm(gm2t_ref[...], csum) * (1.0 / n2)
        var2 = _mm(gm2t_ref[...], cssq) * (1.0 / n2) - mu2 * mu2
        rs2 = jax.lax.rsqrt(var2 + _EPS)
        mu2c = _mm(gm2_ref[...], mu2)
        rs2c = _mm(gm2_ref[...], rs2)
        a2c = g2w_ref[...] * rs2c
        b2c = g2b_ref[...] - mu2c * a2c
        a2 = h2 * a2c[:, :, None] + b2c[:, :, None]
        a2 = a2 * (0.5 * jnp.tanh(a2 * 0.5) + 0.5)
        logits = jax.lax.dot_general(
            w2_ref[...], a2, (((1,), (0,)), ((), ())),
            preferred_element_type=jnp.float32) + b2_ref[...][:, :, None]
        m = jnp.max(logits, axis=0, keepdims=True)
        e = jnp.exp(logits - m)
        sm = e / jnp.sum(e, axis=0, keepdims=True)
        fmask = (jax.lax.broadcasted_iota(jnp.int32, (1, 1, _WPAD), 2)
                 < _WP).astype(jnp.float32)
        pooled = (jnp.sum(sm * fmask, axis=(1, 2)) * (1.0 / _HW)
                  ).reshape(_E, 1)

        io = jax.lax.broadcasted_iota(jnp.int32, (_E, 1), 0)
        m1 = jnp.max(pooled)
        i1 = jnp.min(jnp.where(pooled == m1, io, _E))
        p2 = jnp.where(io == i1, -jnp.inf, pooled)
        m2 = jnp.max(p2)
        i2 = jnp.min(jnp.where(p2 == m2, io, _E))
        den = m1 + m2 + 1e-9
        io2 = jax.lax.broadcasted_iota(jnp.int32, (1, 1, 2), 2)
        vals_ref[...] = jnp.where(io2 == 0, m1 / den, m2 / den)
        idx_ref[...] = jnp.where(io2 == 0, i1, i2)


def kernel(x, dw_w, gn1_w, gn1_b, pw1_w, gn2_w, gn2_b, pw2_w, pw2_b):
    f32 = jnp.float32
    dw2 = dw_w.reshape(_C, 9)
    k = jnp.arange(_W)
    t = jnp.arange(3 * _WPAD) // _WPAD
    j = jnp.arange(3 * _WPAD) % _WPAD
    jj = j[None, :] + (t[None, :] - 1)
    valid = (j[None, :] < _WP) & (jj >= 0) & (jj < _WP)
    pool_mat = ((k[:, None] // 2 == jj) & valid).astype(f32) * 0.25
    row_mat = (jnp.arange(_H)[None, :] // 2 == jnp.arange(_HP)[:, None]).astype(f32)

    h, sums, ssqs = pl.pallas_call(
        _pass1_body,
        grid=(_B, _C // _CB),
        in_specs=[
            pl.BlockSpec((1, _CB, _H, _W), lambda b, c: (b, c, 0, 0)),
            pl.BlockSpec((_CB, 9), lambda b, c: (c, 0)),
            pl.BlockSpec((_HP, _H), lambda b, c: (0, 0)),
            pl.BlockSpec((_W, 3 * _WPAD), lambda b, c: (0, 0)),
        ],
        out_specs=[
            pl.BlockSpec((1, _CB, _HP, _WPAD), lambda b, c: (b, c, 0, 0)),
            pl.BlockSpec((1, _C, 1), lambda b, c: (b, 0, 0)),
            pl.BlockSpec((1, _C, 1), lambda b, c: (b, 0, 0)),
        ],
        out_shape=[
            jax.ShapeDtypeStruct((_B, _C, _HP, _WPAD), jnp.bfloat16),
            jax.ShapeDtypeStruct((_B, _C, 1), f32),
            jax.ShapeDtypeStruct((_B, _C, 1), f32),
        ],
        compiler_params=pltpu.CompilerParams(
            dimension_semantics=("arbitrary", "arbitrary")),
    )(x.astype(jnp.bfloat16), dw2,
      row_mat.astype(jnp.bfloat16), pool_mat.astype(jnp.bfloat16))

    gm1 = (jnp.arange(_C)[:, None] * _G1 // _C == jnp.arange(_G1)[None, :]).astype(f32)
    gm2 = (jnp.arange(_RED)[:, None] * _G2 // _RED == jnp.arange(_G2)[None, :]).astype(f32)

    full = lambda shape: pl.BlockSpec(shape, lambda b, s: tuple(0 for _ in shape))
    _, _, vals, idx = pl.pallas_call(
        _pass2_body,
        grid=(_B, _NS),
        in_specs=[
            pl.BlockSpec((1, _C, _HP // _NS, _WPAD), lambda b, s: (b, 0, s, 0)),
            pl.BlockSpec((1, _C, 1), lambda b, s: (b, 0, 0)),
            pl.BlockSpec((1, _C, 1), lambda b, s: (b, 0, 0)),
            full((_C, 1)), full((_C, 1)), full((_RED, _C)),
            full((_C, _G1)), full((_G1, _C)),
            full((_RED, 1)), full((_RED, 1)), full((_E, _RED)), full((_E, 1)),
            full((_RED, _G2)), full((_G2, _RED)),
        ],
        out_specs=[
            pl.BlockSpec((1, _RED, _HP, _WPAD), lambda b, s: (b, 0, 0, 0)),
            pl.BlockSpec((1, _C, 2), lambda b, s: (b, 0, 0)),
            pl.BlockSpec((1, 1, 2), lambda b, s: (b, 0, 0)),
            pl.BlockSpec((1, 1, 2), lambda b, s: (b, 0, 0)),
        ],
        out_shape=[
            jax.ShapeDtypeStruct((_B, _RED, _HP, _WPAD), f32),
            jax.ShapeDtypeStruct((_B, _C, 2), f32),
            jax.ShapeDtypeStruct((_B, 1, 2), f32),
            jax.ShapeDtypeStruct((_B, 1, 2), jnp.int32),
        ],
        compiler_params=pltpu.CompilerParams(
            dimension_semantics=("arbitrary", "arbitrary")),
    )(h, sums, ssqs,
      gn1_w.reshape(_C, 1), gn1_b.reshape(_C, 1),
      pw1_w.reshape(_RED, _C),
      gm1, gm1.T, gn2_w.reshape(_RED, 1), gn2_b.reshape(_RED, 1),
      pw2_w.reshape(_E, _RED), pw2_b.reshape(_E, 1),
      gm2, gm2.T)

    return vals.reshape(_B, 2, 1, 1), idx.reshape(_B, 2, 1, 1)


# speedup vs baseline: 1.0574x; 1.0574x over previous
import jax
import jax.numpy as jnp
from jax.experimental import pallas as pl
from jax.experimental.pallas import tpu as pltpu

_B, _C, _H, _W = 4, 768, 224, 224
_HP, _WP = _H // 2, _W // 2
_WPAD = 128
_HWP = _HP * _WPAD
_HW = _HP * _WP
_E, _RED = 16, 48
_G1, _G2 = 8, 4
_EPS = 1e-5
_CB = 16
_S = _HWP // 7
_NS = _HWP // _S


def _pass1_body(x_ref, dw_ref, ph_ref, pw_ref, h_ref, sum_ref, ssq_ref):
    xb = x_ref[0].astype(jnp.bfloat16)
    phb = jnp.broadcast_to(ph_ref[...], (_CB, _HP, _H))
    xh = jax.lax.dot_general(
        phb, xb, (((2,), (1,)), ((0,), (0,))),
        preferred_element_type=jnp.float32,
    ).astype(jnp.bfloat16)
    r = jax.lax.dot_general(
        xh.reshape(_CB * _HP, _W), pw_ref[...],
        (((1,), (0,)), ((), ())),
        preferred_element_type=jnp.float32,
    ).reshape(_CB, _HP, 3 * _WPAD)

    w = dw_ref[...]
    rows = []
    for dy in (-1, 0, 1):
        acc = jnp.zeros((_CB, _HP, _WPAD), jnp.float32)
        for t in range(3):
            k = 3 * (dy + 1) + t
            q = r[:, :, t * _WPAD:(t + 1) * _WPAD]
            acc = acc + q * w[:, k:k + 1].reshape(_CB, 1, 1)
        rows.append(acc)

    ri = jax.lax.broadcasted_iota(jnp.int32, (1, _HP, 1), 1)
    h = rows[1]
    h = h + jnp.roll(rows[0], 1, axis=1) * (ri >= 1).astype(jnp.float32)
    h = h + jnp.roll(rows[2], -1, axis=1) * (ri < _HP - 1).astype(jnp.float32)

    h_ref[0] = h.astype(jnp.bfloat16)
    cb = pl.program_id(1)
    sum_ref[0, pl.ds(cb * _CB, _CB), :] = jnp.sum(h, axis=(1, 2)).reshape(_CB, 1)
    ssq_ref[0, pl.ds(cb * _CB, _CB), :] = jnp.sum(h * h, axis=(1, 2)).reshape(_CB, 1)


def _pass2_body(h_ref, sum_ref, ssq_ref, g1w_ref, g1b_ref, w1_ref,
                gm1_ref, gm1t_ref, g2w_ref, g2b_ref, w2_ref, b2_ref,
                gm2_ref, gm2t_ref, h2_ref, coef_ref, vals_ref, idx_ref):
    s = pl.program_id(1)

    def _mm(a, b):
        return jax.lax.dot_general(a, b, (((1,), (0,)), ((), ())),
                                   preferred_element_type=jnp.float32)

    @pl.when(s == 0)
    def _():
        n1 = float((_C // _G1) * _HW)
        mu_g = _mm(gm1t_ref[...], sum_ref[0]) * (1.0 / n1)
        var_g = _mm(gm1t_ref[...], ssq_ref[0]) * (1.0 / n1) - mu_g * mu_g
        rs_g = jax.lax.rsqrt(var_g + _EPS)
        mu_c = _mm(gm1_ref[...], mu_g)
        rs_c = _mm(gm1_ref[...], rs_g)
        a_c = g1w_ref[...] * rs_c
        b_c = g1b_ref[...] - mu_c * a_c
        coef_ref[0] = jnp.concatenate([a_c, b_c], axis=1)

    ab = coef_ref[0]
    a_c = ab[:, 0:1, None]
    b_c = ab[:, 1:2, None]
    h = h_ref[0].astype(jnp.float32)
    a1 = h * a_c + b_c
    a1 = a1 * (0.5 * jnp.tanh(a1 * 0.5) + 0.5)
    li = jax.lax.broadcasted_iota(jnp.int32, (1, 1, _WPAD), 2)
    cmask = (li < _WP).astype(jnp.float32)
    chunk = jax.lax.dot_general(w1_ref[...], a1, (((1,), (0,)), ((), ())),
                                preferred_element_type=jnp.float32)
    _RB = _HP // _NS
    h2_ref[0, :, pl.ds(s * _RB, _RB), :] = chunk * cmask

    @pl.when(s == _NS - 1)
    def _():
        h2 = h2_ref[0]
        n2 = float((_RED // _G2) * _HW)
        csum = jnp.sum(h2, axis=(1, 2)).reshape(_RED, 1)
        cssq = jnp.sum(h2 * h2, axis=(1, 2)).reshape(_RED, 1)
        mu2 = _mm(gm2t_ref[...], csum) * (1.0 / n2)
        var2 = _mm(gm2t_ref[...], cssq) * (1.0 / n2) - mu2 * mu2
        rs2 = jax.lax.rsqrt(var2 + _EPS)
        mu2c = _mm(gm2_ref[...], mu2)
        rs2c = _mm(gm2_ref[...], rs2)
        a2c = g2w_ref[...] * rs2c
        b2c = g2b_ref[...] - mu2c * a2c
        a2 = h2 * a2c[:, :, None] + b2c[:, :, None]
        a2 = a2 * (0.5 * jnp.tanh(a2 * 0.5) + 0.5)
        logits = jax.lax.dot_general(
            w2_ref[...], a2, (((1,), (0,)), ((), ())),
            preferred_element_type=jnp.float32) + b2_ref[...][:, :, None]
        m = jnp.max(logits, axis=0, keepdims=True)
        e = jnp.exp(logits - m)
        sm = e / jnp.sum(e, axis=0, keepdims=True)
        fmask = (jax.lax.broadcasted_iota(jnp.int32, (1, 1, _WPAD), 2)
                 < _WP).astype(jnp.float32)
        pooled = (jnp.sum(sm * fmask, axis=(1, 2)) * (1.0 / _HW)
                  ).reshape(_E, 1)

        io = jax.lax.broadcasted_iota(jnp.int32, (_E, 1), 0)
        m1 = jnp.max(pooled)
        i1 = jnp.min(jnp.where(pooled == m1, io, _E))
        p2 = jnp.where(io == i1, -jnp.inf, pooled)
        m2 = jnp.max(p2)
        i2 = jnp.min(jnp.where(p2 == m2, io, _E))
        den = m1 + m2 + 1e-9
        io2 = jax.lax.broadcasted_iota(jnp.int32, (1, 1, 2), 2)
        vals_ref[...] = jnp.where(io2 == 0, m1 / den, m2 / den)
        idx_ref[...] = jnp.where(io2 == 0, i1, i2)


def kernel(x, dw_w, gn1_w, gn1_b, pw1_w, gn2_w, gn2_b, pw2_w, pw2_b):
    f32 = jnp.float32
    dw2 = dw_w.reshape(_C, 9)
    k = jnp.arange(_W)
    t = jnp.arange(3 * _WPAD) // _WPAD
    j = jnp.arange(3 * _WPAD) % _WPAD
    jj = j[None, :] + (t[None, :] - 1)
    valid = (j[None, :] < _WP) & (jj >= 0) & (jj < _WP)
    pool_mat = ((k[:, None] // 2 == jj) & valid).astype(f32) * 0.25
    row_mat = (jnp.arange(_H)[None, :] // 2 == jnp.arange(_HP)[:, None]).astype(f32)

    h, sums, ssqs = pl.pallas_call(
        _pass1_body,
        grid=(_B, _C // _CB),
        in_specs=[
            pl.BlockSpec((1, _CB, _H, _W), lambda b, c: (b, c, 0, 0)),
            pl.BlockSpec((_CB, 9), lambda b, c: (c, 0)),
            pl.BlockSpec((_HP, _H), lambda b, c: (0, 0)),
            pl.BlockSpec((_W, 3 * _WPAD), lambda b, c: (0, 0)),
        ],
        out_specs=[
            pl.BlockSpec((1, _CB, _HP, _WPAD), lambda b, c: (b, c, 0, 0)),
            pl.BlockSpec((1, _C, 1), lambda b, c: (b, 0, 0)),
            pl.BlockSpec((1, _C, 1), lambda b, c: (b, 0, 0)),
        ],
        out_shape=[
            jax.ShapeDtypeStruct((_B, _C, _HP, _WPAD), jnp.bfloat16),
            jax.ShapeDtypeStruct((_B, _C, 1), f32),
            jax.ShapeDtypeStruct((_B, _C, 1), f32),
        ],
        compiler_params=pltpu.CompilerParams(
            dimension_semantics=("arbitrary", "arbitrary")),
    )(x, dw2, row_mat.astype(jnp.bfloat16), pool_mat.astype(jnp.bfloat16))

    gm1 = (jnp.arange(_C)[:, None] * _G1 // _C == jnp.arange(_G1)[None, :]).astype(f32)
    gm2 = (jnp.arange(_RED)[:, None] * _G2 // _RED == jnp.arange(_G2)[None, :]).astype(f32)

    full = lambda shape: pl.BlockSpec(shape, lambda b, s: tuple(0 for _ in shape))
    _, _, vals, idx = pl.pallas_call(
        _pass2_body,
        grid=(_B, _NS),
        in_specs=[
            pl.BlockSpec((1, _C, _HP // _NS, _WPAD), lambda b, s: (b, 0, s, 0)),
            pl.BlockSpec((1, _C, 1), lambda b, s: (b, 0, 0)),
            pl.BlockSpec((1, _C, 1), lambda b, s: (b, 0, 0)),
            full((_C, 1)), full((_C, 1)), full((_RED, _C)),
            full((_C, _G1)), full((_G1, _C)),
            full((_RED, 1)), full((_RED, 1)), full((_E, _RED)), full((_E, 1)),
            full((_RED, _G2)), full((_G2, _RED)),
        ],
        out_specs=[
            pl.BlockSpec((1, _RED, _HP, _WPAD), lambda b, s: (b, 0, 0, 0)),
            pl.BlockSpec((1, _C, 2), lambda b, s: (b, 0, 0)),
            pl.BlockSpec((1, 1, 2), lambda b, s: (b, 0, 0)),
            pl.BlockSpec((1, 1, 2), lambda b, s: (b, 0, 0)),
        ],
        out_shape=[
            jax.ShapeDtypeStruct((_B, _RED, _HP, _WPAD), f32),
            jax.ShapeDtypeStruct((_B, _C, 2), f32),
            jax.ShapeDtypeStruct((_B, 1, 2), f32),
            jax.ShapeDtypeStruct((_B, 1, 2), jnp.int32),
        ],
        compiler_params=pltpu.CompilerParams(
            dimension_semantics=("arbitrary", "arbitrary")),
    )(h, sums, ssqs,
      gn1_w.reshape(_C, 1), gn1_b.reshape(_C, 1),
      pw1_w.reshape(_RED, _C),
      gm1, gm1.T, gn2_w.reshape(_RED, 1), gn2_b.reshape(_RED, 1),
      pw2_w.reshape(_E, _RED), pw2_b.reshape(_E, 1),
      gm2, gm2.T)

    return vals.reshape(_B, 2, 1, 1), idx.reshape(_B, 2, 1, 1)


# speedup vs baseline: 1.1135x; 1.0531x over previous
import jax
import jax.numpy as jnp
from jax.experimental import pallas as pl
from jax.experimental.pallas import tpu as pltpu

_B, _C, _H, _W = 4, 768, 224, 224
_HP, _WP = _H // 2, _W // 2
_WPAD = 128
_HWP = _HP * _WPAD
_HW = _HP * _WP
_E, _RED = 16, 48
_G1, _G2 = 8, 4
_EPS = 1e-5
_CB = 32
_S = _HWP // 7
_NS = _HWP // _S


def _pass1_body(x_ref, dw_ref, ph_ref, pw_ref, h_ref, sum_ref, ssq_ref):
    xb = x_ref[0].astype(jnp.bfloat16)
    phb = jnp.broadcast_to(ph_ref[...], (_CB, _HP, _H))
    xh = jax.lax.dot_general(
        phb, xb, (((2,), (1,)), ((0,), (0,))),
        preferred_element_type=jnp.float32,
    ).astype(jnp.bfloat16)
    r = jax.lax.dot_general(
        xh.reshape(_CB * _HP, _W), pw_ref[...],
        (((1,), (0,)), ((), ())),
        preferred_element_type=jnp.float32,
    ).reshape(_CB, _HP, 3 * _WPAD)

    w = dw_ref[...]
    rows = []
    for dy in (-1, 0, 1):
        acc = jnp.zeros((_CB, _HP, _WPAD), jnp.float32)
        for t in range(3):
            k = 3 * (dy + 1) + t
            q = r[:, :, t * _WPAD:(t + 1) * _WPAD]
            acc = acc + q * w[:, k:k + 1].reshape(_CB, 1, 1)
        rows.append(acc)

    ri = jax.lax.broadcasted_iota(jnp.int32, (1, _HP, 1), 1)
    h = rows[1]
    h = h + jnp.roll(rows[0], 1, axis=1) * (ri >= 1).astype(jnp.float32)
    h = h + jnp.roll(rows[2], -1, axis=1) * (ri < _HP - 1).astype(jnp.float32)

    h_ref[0] = h.astype(jnp.bfloat16)
    cb = pl.program_id(1)
    sum_ref[0, pl.ds(cb * _CB, _CB), :] = jnp.sum(h, axis=(1, 2)).reshape(_CB, 1)
    ssq_ref[0, pl.ds(cb * _CB, _CB), :] = jnp.sum(h * h, axis=(1, 2)).reshape(_CB, 1)


def _pass2_body(h_ref, sum_ref, ssq_ref, g1w_ref, g1b_ref, w1_ref,
                gm1_ref, gm1t_ref, g2w_ref, g2b_ref, w2_ref, b2_ref,
                gm2_ref, gm2t_ref, h2_ref, coef_ref, vals_ref, idx_ref):
    s = pl.program_id(1)

    def _mm(a, b):
        return jax.lax.dot_general(a, b, (((1,), (0,)), ((), ())),
                                   preferred_element_type=jnp.float32)

    @pl.when(s == 0)
    def _():
        n1 = float((_C // _G1) * _HW)
        mu_g = _mm(gm1t_ref[...], sum_ref[0]) * (1.0 / n1)
        var_g = _mm(gm1t_ref[...], ssq_ref[0]) * (1.0 / n1) - mu_g * mu_g
        rs_g = jax.lax.rsqrt(var_g + _EPS)
        mu_c = _mm(gm1_ref[...], mu_g)
        rs_c = _mm(gm1_ref[...], rs_g)
        a_c = g1w_ref[...] * rs_c
        b_c = g1b_ref[...] - mu_c * a_c
        coef_ref[0] = jnp.concatenate([a_c, b_c], axis=1)

    ab = coef_ref[0]
    a_c = ab[:, 0:1, None]
    b_c = ab[:, 1:2, None]
    h = h_ref[0].astype(jnp.float32)
    a1 = h * a_c + b_c
    a1 = a1 * (0.5 * jnp.tanh(a1 * 0.5) + 0.5)
    li = jax.lax.broadcasted_iota(jnp.int32, (1, 1, _WPAD), 2)
    cmask = (li < _WP).astype(jnp.float32)
    chunk = jax.lax.dot_general(w1_ref[...], a1, (((1,), (0,)), ((), ())),
                                preferred_element_type=jnp.float32)
    _RB = _HP // _NS
    h2_ref[0, :, pl.ds(s * _RB, _RB), :] = chunk * cmask

    @pl.when(s == _NS - 1)
    def _():
        h2 = h2_ref[0]
        n2 = float((_RED // _G2) * _HW)
        csum = jnp.sum(h2, axis=(1, 2)).reshape(_RED, 1)
        cssq = jnp.sum(h2 * h2, axis=(1, 2)).reshape(_RED, 1)
        mu2 = _mm(gm2t_ref[...], csum) * (1.0 / n2)
        var2 = _mm(gm2t_ref[...], cssq) * (1.0 / n2) - mu2 * mu2
        rs2 = jax.lax.rsqrt(var2 + _EPS)
        mu2c = _mm(gm2_ref[...], mu2)
        rs2c = _mm(gm2_ref[...], rs2)
        a2c = g2w_ref[...] * rs2c
        b2c = g2b_ref[...] - mu2c * a2c
        a2 = h2 * a2c[:, :, None] + b2c[:, :, None]
        a2 = a2 * (0.5 * jnp.tanh(a2 * 0.5) + 0.5)
        logits = jax.lax.dot_general(
            w2_ref[...], a2, (((1,), (0,)), ((), ())),
            preferred_element_type=jnp.float32) + b2_ref[...][:, :, None]
        m = jnp.max(logits, axis=0, keepdims=True)
        e = jnp.exp(logits - m)
        sm = e / jnp.sum(e, axis=0, keepdims=True)
        fmask = (jax.lax.broadcasted_iota(jnp.int32, (1, 1, _WPAD), 2)
                 < _WP).astype(jnp.float32)
        pooled = (jnp.sum(sm * fmask, axis=(1, 2)) * (1.0 / _HW)
                  ).reshape(_E, 1)

        io = jax.lax.broadcasted_iota(jnp.int32, (_E, 1), 0)
        m1 = jnp.max(pooled)
        i1 = jnp.min(jnp.where(pooled == m1, io, _E))
        p2 = jnp.where(io == i1, -jnp.inf, pooled)
        m2 = jnp.max(p2)
        i2 = jnp.min(jnp.where(p2 == m2, io, _E))
        den = m1 + m2 + 1e-9
        io2 = jax.lax.broadcasted_iota(jnp.int32, (1, 1, 2), 2)
        vals_ref[...] = jnp.where(io2 == 0, m1 / den, m2 / den)
        idx_ref[...] = jnp.where(io2 == 0, i1, i2)


def kernel(x, dw_w, gn1_w, gn1_b, pw1_w, gn2_w, gn2_b, pw2_w, pw2_b):
    f32 = jnp.float32
    dw2 = dw_w.reshape(_C, 9)
    k = jnp.arange(_W)
    t = jnp.arange(3 * _WPAD) // _WPAD
    j = jnp.arange(3 * _WPAD) % _WPAD
    jj = j[None, :] + (t[None, :] - 1)
    valid = (j[None, :] < _WP) & (jj >= 0) & (jj < _WP)
    pool_mat = ((k[:, None] // 2 == jj) & valid).astype(f32) * 0.25
    row_mat = (jnp.arange(_H)[None, :] // 2 == jnp.arange(_HP)[:, None]).astype(f32)

    h, sums, ssqs = pl.pallas_call(
        _pass1_body,
        grid=(_B, _C // _CB),
        in_specs=[
            pl.BlockSpec((1, _CB, _H, _W), lambda b, c: (b, c, 0, 0)),
            pl.BlockSpec((_CB, 9), lambda b, c: (c, 0)),
            pl.BlockSpec((_HP, _H), lambda b, c: (0, 0)),
            pl.BlockSpec((_W, 3 * _WPAD), lambda b, c: (0, 0)),
        ],
        out_specs=[
            pl.BlockSpec((1, _CB, _HP, _WPAD), lambda b, c: (b, c, 0, 0)),
            pl.BlockSpec((1, _C, 1), lambda b, c: (b, 0, 0)),
            pl.BlockSpec((1, _C, 1), lambda b, c: (b, 0, 0)),
        ],
        out_shape=[
            jax.ShapeDtypeStruct((_B, _C, _HP, _WPAD), jnp.bfloat16),
            jax.ShapeDtypeStruct((_B, _C, 1), f32),
            jax.ShapeDtypeStruct((_B, _C, 1), f32),
        ],
        compiler_params=pltpu.CompilerParams(
            dimension_semantics=("arbitrary", "arbitrary")),
    )(x, dw2, row_mat.astype(jnp.bfloat16), pool_mat.astype(jnp.bfloat16))

    gm1 = (jnp.arange(_C)[:, None] * _G1 // _C == jnp.arange(_G1)[None, :]).astype(f32)
    gm2 = (jnp.arange(_RED)[:, None] * _G2 // _RED == jnp.arange(_G2)[None, :]).astype(f32)

    full = lambda shape: pl.BlockSpec(shape, lambda b, s: tuple(0 for _ in shape))
    _, _, vals, idx = pl.pallas_call(
        _pass2_body,
        grid=(_B, _NS),
        in_specs=[
            pl.BlockSpec((1, _C, _HP // _NS, _WPAD), lambda b, s: (b, 0, s, 0)),
            pl.BlockSpec((1, _C, 1), lambda b, s: (b, 0, 0)),
            pl.BlockSpec((1, _C, 1), lambda b, s: (b, 0, 0)),
            full((_C, 1)), full((_C, 1)), full((_RED, _C)),
            full((_C, _G1)), full((_G1, _C)),
            full((_RED, 1)), full((_RED, 1)), full((_E, _RED)), full((_E, 1)),
            full((_RED, _G2)), full((_G2, _RED)),
        ],
        out_specs=[
            pl.BlockSpec((1, _RED, _HP, _WPAD), lambda b, s: (b, 0, 0, 0)),
            pl.BlockSpec((1, _C, 2), lambda b, s: (b, 0, 0)),
            pl.BlockSpec((1, 1, 2), lambda b, s: (b, 0, 0)),
            pl.BlockSpec((1, 1, 2), lambda b, s: (b, 0, 0)),
        ],
        out_shape=[
            jax.ShapeDtypeStruct((_B, _RED, _HP, _WPAD), f32),
            jax.ShapeDtypeStruct((_B, _C, 2), f32),
            jax.ShapeDtypeStruct((_B, 1, 2), f32),
            jax.ShapeDtypeStruct((_B, 1, 2), jnp.int32),
        ],
        compiler_params=pltpu.CompilerParams(
            dimension_semantics=("arbitrary", "arbitrary")),
    )(h, sums, ssqs,
      gn1_w.reshape(_C, 1), gn1_b.reshape(_C, 1),
      pw1_w.reshape(_RED, _C),
      gm1, gm1.T, gn2_w.reshape(_RED, 1), gn2_b.reshape(_RED, 1),
      pw2_w.reshape(_E, _RED), pw2_b.reshape(_E, 1),
      gm2, gm2.T)

    return vals.reshape(_B, 2, 1, 1), idx.reshape(_B, 2, 1, 1)
